# baseline (speedup 1.0000x reference)
import functools

import jax
import jax.numpy as jnp
from jax import lax
from jax.experimental import pallas as pl
from jax.experimental.pallas import tpu as pltpu
from jax.experimental.pallas import tpu_sc as plsc

NC = 2
NS = 16


def _make_deg(n_pad, cpt):
    rows_per_tile = n_pad // NS
    mesh = plsc.VectorSubcoreMesh(core_axis_name="c", subcore_axis_name="s")

    @functools.partial(
        pl.kernel,
        out_type=jax.ShapeDtypeStruct((n_pad, 16), jnp.float32),
        mesh=mesh,
        scratch_types=[
            pltpu.VMEM((cpt, 128), jnp.int32),
            pltpu.VMEM((128, 16), jnp.float32),
            pltpu.VMEM_SHARED((n_pad, 16), jnp.float32),
            pltpu.SemaphoreType.DMA,
        ],
        compiler_params=pltpu.CompilerParams(use_tc_tiling_on_sc=False),
    )
    def deg_kernel(dst_hbm, ones_hbm, zeros_hbm, out_hbm, dst_v, ones_v, acc,
                   ssem):
        c = lax.axis_index("c")
        s = lax.axis_index("s")

        @pl.when(c == 0)
        def _():
            r0 = s * rows_per_tile
            pltpu.sync_copy(zeros_hbm.at[pl.ds(r0, rows_per_tile)],
                            acc.at[pl.ds(r0, rows_per_tile)])
            pltpu.sync_copy(dst_hbm.at[pl.ds(s * cpt, cpt)], dst_v)
            pltpu.sync_copy(ones_hbm, ones_v)
            plsc.subcore_barrier()

            def body(j, carry):
                pltpu.async_copy(ones_v, acc.at[dst_v.at[j]], ssem, add=True)
                return carry

            lax.fori_loop(0, cpt, body, 0)

            def drain(j, carry):
                pltpu.make_async_copy(ones_v, acc.at[dst_v.at[j]], ssem).wait()
                return carry

            lax.fori_loop(0, cpt, drain, 0)
            plsc.subcore_barrier()
            pltpu.sync_copy(acc.at[pl.ds(r0, rows_per_tile)],
                            out_hbm.at[pl.ds(r0, rows_per_tile)])

    return deg_kernel


def _make_agg(n_pad, width, ch, cpt):
    rows_per_tile = n_pad // NS
    mesh = plsc.VectorSubcoreMesh(core_axis_name="c", subcore_axis_name="s")

    nbuf = 8
    assert cpt % nbuf == 0
    rounds = cpt // nbuf

    @functools.partial(
        pl.kernel,
        out_type=jax.ShapeDtypeStruct((n_pad, width), jnp.float32),
        mesh=mesh,
        scratch_types=[
            pltpu.VMEM((cpt, ch), jnp.int32),
            pltpu.VMEM((cpt, ch), jnp.int32),
            pltpu.VMEM((nbuf, ch, width), jnp.float32),
            pltpu.VMEM_SHARED((n_pad, width), jnp.float32),
            pltpu.SemaphoreType.DMA((nbuf,)),
            pltpu.SemaphoreType.DMA((nbuf,)),
        ],
        compiler_params=pltpu.CompilerParams(use_tc_tiling_on_sc=False),
    )
    def agg_kernel(table_hbm, src_hbm, dst_hbm, zeros_hbm, out_hbm,
                   src_v, dst_v, rows_v, acc, gsem, ssem):
        c = lax.axis_index("c")
        s = lax.axis_index("s")

        @pl.when(c == 0)
        def _():
            r0 = s * rows_per_tile
            pltpu.sync_copy(src_hbm.at[pl.ds(s * cpt, cpt)], src_v)
            pltpu.sync_copy(dst_hbm.at[pl.ds(s * cpt, cpt)], dst_v)
            for b in range(nbuf):
                pltpu.async_copy(table_hbm.at[src_v.at[b]], rows_v.at[b],
                                 gsem.at[b])
            pltpu.sync_copy(zeros_hbm.at[pl.ds(r0, rows_per_tile)],
                            acc.at[pl.ds(r0, rows_per_tile)])
            plsc.subcore_barrier()

            def body(g, carry):
                base = g * nbuf
                for b in range(nbuf):
                    pltpu.make_async_copy(table_hbm.at[src_v.at[base + b]],
                                          rows_v.at[b], gsem.at[b]).wait()
                    pltpu.async_copy(rows_v.at[b], acc.at[dst_v.at[base + b]],
                                     ssem.at[b], add=True)
                for b in range(nbuf):
                    pltpu.make_async_copy(rows_v.at[b],
                                          acc.at[dst_v.at[base + b]],
                                          ssem.at[b]).wait()
                    pltpu.async_copy(table_hbm.at[src_v.at[base + nbuf + b]],
                                     rows_v.at[b], gsem.at[b])
                return carry

            lax.fori_loop(0, rounds - 1, body, 0)

            base = (rounds - 1) * nbuf
            for b in range(nbuf):
                pltpu.make_async_copy(table_hbm.at[src_v.at[base + b]],
                                      rows_v.at[b], gsem.at[b]).wait()
                pltpu.async_copy(rows_v.at[b], acc.at[dst_v.at[base + b]],
                                 ssem.at[b], add=True)
            for b in range(nbuf):
                pltpu.make_async_copy(rows_v.at[b], acc.at[dst_v.at[base + b]],
                                      ssem.at[b]).wait()
            plsc.subcore_barrier()
            pltpu.sync_copy(acc.at[pl.ds(r0, rows_per_tile)],
                            out_hbm.at[pl.ds(r0, rows_per_tile)])

    return agg_kernel


def _mm0_body(x_ref, w_ref, deg_ref, y_ref, dinv_ref):
    deg = deg_ref[:, 0:1] + 1.0
    dinv = lax.rsqrt(deg)
    y = jnp.dot(x_ref[:], w_ref[:], preferred_element_type=jnp.float32)
    y_ref[:] = y * dinv
    dinv_ref[:] = dinv


def _mid_body(p_ref, y0_ref, dinv_ref, b_ref, w_ref, out_ref):
    dinv = dinv_ref[:]
    pre = (p_ref[:] + y0_ref[:]) * dinv + b_ref[:]
    h = jnp.where(pre >= 0, pre, 0.01 * pre)
    out_ref[:] = jnp.dot(h, w_ref[:], preferred_element_type=jnp.float32) * dinv


def _make_final_body(num_graphs, n, bn):
    def final_body(p_ref, y1_ref, dinv_ref, b_ref, bat_ref, out_ref):
        i = pl.program_id(0)

        @pl.when(i == 0)
        def _():
            out_ref[:] = jnp.full(out_ref.shape, -jnp.inf, jnp.float32)

        pre = (p_ref[:] + y1_ref[:]) * dinv_ref[:] + b_ref[:]
        h = jnp.where(pre >= 0, pre, 0.01 * pre)
        rid = jax.lax.broadcasted_iota(jnp.int32, (bn, 1), 0) + i * bn
        h = jnp.where(rid < n, h, -jnp.inf)
        bat = jnp.where(rid < n, bat_ref[:], num_graphs - 1)
        g_lo = jnp.clip(bat[0, 0], 0, num_graphs - 1)
        g_hi = jnp.clip(bat[bn - 1, 0], g_lo, num_graphs - 1)

        def body(g, carry):
            m = jnp.max(jnp.where(bat == g, h, -jnp.inf), axis=0,
                        keepdims=True)
            out_ref[pl.ds(g, 1), :] = jnp.maximum(out_ref[pl.ds(g, 1), :], m)
            return carry

        lax.fori_loop(g_lo, g_hi + 1, body, 0)

    return final_body


def _row_spec(bn, width):
    return pl.BlockSpec((bn, width), lambda i: (i, 0))


def _full_spec(shape):
    return pl.BlockSpec(shape, lambda i: tuple(0 for _ in shape))


def kernel(x, edge_index, batch, W0, b0, W1, b1):
    n, in_ch = x.shape
    hid = W0.shape[1]
    out_ch = W1.shape[1]
    e = edge_index.shape[1]
    num_graphs = 64

    bn = 1024
    n_pad = ((n + 1 + NS * 16 - 1) // (NS * 16)) * (NS * 16)
    n_pad = ((n_pad + bn - 1) // bn) * bn
    e_pad = ((e + NS * 8 * 128 - 1) // (NS * 8 * 128)) * (NS * 8 * 128)
    cpt_128 = e_pad // (128 * NS)
    cpt_64 = e_pad // (64 * NS)

    pad = jnp.full((e_pad - e,), n, jnp.int32)
    src_flat = jnp.concatenate([edge_index[0], pad])
    dst_flat = jnp.concatenate([edge_index[1], pad])
    src64 = src_flat.reshape(e_pad // 64, 64)
    dst64 = dst_flat.reshape(e_pad // 64, 64)
    dst128 = dst_flat.reshape(e_pad // 128, 128)

    ones_row = jnp.ones((128, 16), jnp.float32)
    z16 = jnp.zeros((n_pad, 16), jnp.float32)
    zh = jnp.zeros((n_pad, hid), jnp.float32)
    zo = jnp.zeros((n_pad, out_ch), jnp.float32)

    deg16 = _make_deg(n_pad, cpt_128)(dst128, ones_row, z16)

    grid = (n_pad // bn,)

    y0, dinv = pl.pallas_call(
        _mm0_body,
        grid=grid,
        in_specs=[
            _row_spec(bn, in_ch),
            _full_spec((in_ch, hid)),
            _row_spec(bn, 16),
        ],
        out_specs=[_row_spec(bn, hid), _row_spec(bn, 1)],
        out_shape=[
            jax.ShapeDtypeStruct((n_pad, hid), jnp.float32),
            jax.ShapeDtypeStruct((n_pad, 1), jnp.float32),
        ],
    )(x, W0, deg16)

    p = _make_agg(n_pad, hid, 64, cpt_64)(y0, src64, dst64, zh)

    y1 = pl.pallas_call(
        _mid_body,
        grid=grid,
        in_specs=[
            _row_spec(bn, hid),
            _row_spec(bn, hid),
            _row_spec(bn, 1),
            _full_spec((1, hid)),
            _full_spec((hid, out_ch)),
        ],
        out_specs=_row_spec(bn, out_ch),
        out_shape=jax.ShapeDtypeStruct((n_pad, out_ch), jnp.float32),
    )(p, y0, dinv, b0[None, :], W1)

    p2 = _make_agg(n_pad, out_ch, 64, cpt_64)(y1, src64, dst64, zo)

    out = pl.pallas_call(
        _make_final_body(num_graphs, n, bn),
        grid=grid,
        in_specs=[
            _row_spec(bn, out_ch),
            _row_spec(bn, out_ch),
            _row_spec(bn, 1),
            _full_spec((1, out_ch)),
            _row_spec(bn, 1),
        ],
        out_specs=_full_spec((num_graphs, out_ch)),
        out_shape=jax.ShapeDtypeStruct((num_graphs, out_ch), jnp.float32),
    )(p2, y1, dinv, b1[None, :], batch[:, None])

    return out


# speedup vs baseline: 1.1244x; 1.1244x over previous
import functools

import jax
import jax.numpy as jnp
from jax import lax
from jax.experimental import pallas as pl
from jax.experimental.pallas import tpu as pltpu
from jax.experimental.pallas import tpu_sc as plsc

NC = 2
NS = 16


def _make_deg(n_pad, cpt):
    rows_per_tile = n_pad // NS
    mesh = plsc.VectorSubcoreMesh(core_axis_name="c", subcore_axis_name="s")

    @functools.partial(
        pl.kernel,
        out_type=jax.ShapeDtypeStruct((n_pad, 16), jnp.float32),
        mesh=mesh,
        scratch_types=[
            pltpu.VMEM((cpt, 128), jnp.int32),
            pltpu.VMEM((128, 16), jnp.float32),
            pltpu.VMEM_SHARED((n_pad, 16), jnp.float32),
            pltpu.SemaphoreType.DMA,
        ],
        compiler_params=pltpu.CompilerParams(use_tc_tiling_on_sc=False),
    )
    def deg_kernel(dst_hbm, ones_hbm, zeros_hbm, out_hbm, dst_v, ones_v, acc,
                   ssem):
        c = lax.axis_index("c")
        s = lax.axis_index("s")

        @pl.when(c == 0)
        def _():
            r0 = s * rows_per_tile
            pltpu.sync_copy(zeros_hbm.at[pl.ds(r0, rows_per_tile)],
                            acc.at[pl.ds(r0, rows_per_tile)])
            pltpu.sync_copy(dst_hbm.at[pl.ds(s * cpt, cpt)], dst_v)
            pltpu.sync_copy(ones_hbm, ones_v)
            plsc.subcore_barrier()

            def body(j, carry):
                pltpu.async_copy(ones_v, acc.at[dst_v.at[j]], ssem, add=True)
                return carry

            lax.fori_loop(0, cpt, body, 0)

            def drain(j, carry):
                pltpu.make_async_copy(ones_v, acc.at[dst_v.at[j]], ssem).wait()
                return carry

            lax.fori_loop(0, cpt, drain, 0)
            plsc.subcore_barrier()
            pltpu.sync_copy(acc.at[pl.ds(r0, rows_per_tile)],
                            out_hbm.at[pl.ds(r0, rows_per_tile)])

    return deg_kernel


def _make_agg(n_pad, width, ch, cpt0, cpt1):
    rows_per_tile = n_pad // NS
    mesh = plsc.VectorSubcoreMesh(core_axis_name="c", subcore_axis_name="s")

    nbuf = 8
    assert cpt0 % nbuf == 0 and cpt1 % nbuf == 0

    @functools.partial(
        pl.kernel,
        out_type=jax.ShapeDtypeStruct((NC, n_pad, width), jnp.float32),
        mesh=mesh,
        scratch_types=[
            pltpu.VMEM((cpt0, ch), jnp.int32),
            pltpu.VMEM((cpt0, ch), jnp.int32),
            pltpu.VMEM((nbuf, ch, width), jnp.float32),
            pltpu.VMEM_SHARED((n_pad, width), jnp.float32),
            pltpu.SemaphoreType.DMA((nbuf,)),
            pltpu.SemaphoreType.DMA((nbuf,)),
        ],
        compiler_params=pltpu.CompilerParams(use_tc_tiling_on_sc=False),
    )
    def agg_kernel(table_hbm, src_hbm, dst_hbm, zeros_hbm, out_hbm,
                   src_v, dst_v, rows_v, acc, gsem, ssem):
        c = lax.axis_index("c")
        s = lax.axis_index("s")
        rounds = jnp.where(c == 0, cpt0 // nbuf, cpt1 // nbuf)
        start = jnp.where(c == 0, s * cpt0, NS * cpt0 + s * cpt1)
        if True:
            r0 = s * rows_per_tile

            @pl.when(c == 0)
            def _():
                pltpu.sync_copy(src_hbm.at[pl.ds(start, cpt0)], src_v)
                pltpu.sync_copy(dst_hbm.at[pl.ds(start, cpt0)], dst_v)

            @pl.when(c != 0)
            def _():
                pltpu.sync_copy(src_hbm.at[pl.ds(start, cpt1)],
                                src_v.at[pl.ds(0, cpt1)])
                pltpu.sync_copy(dst_hbm.at[pl.ds(start, cpt1)],
                                dst_v.at[pl.ds(0, cpt1)])
            for b in range(nbuf):
                pltpu.async_copy(table_hbm.at[src_v.at[b]], rows_v.at[b],
                                 gsem.at[b])
            pltpu.sync_copy(zeros_hbm.at[pl.ds(r0, rows_per_tile)],
                            acc.at[pl.ds(r0, rows_per_tile)])
            plsc.subcore_barrier()

            def body(g, carry):
                base = g * nbuf
                for b in range(nbuf):
                    pltpu.make_async_copy(table_hbm.at[src_v.at[base + b]],
                                          rows_v.at[b], gsem.at[b]).wait()
                    pltpu.async_copy(rows_v.at[b], acc.at[dst_v.at[base + b]],
                                     ssem.at[b], add=True)
                for b in range(nbuf):
                    pltpu.make_async_copy(rows_v.at[b],
                                          acc.at[dst_v.at[base + b]],
                                          ssem.at[b]).wait()
                    pltpu.async_copy(table_hbm.at[src_v.at[base + nbuf + b]],
                                     rows_v.at[b], gsem.at[b])
                return carry

            lax.fori_loop(0, rounds - 1, body, 0)

            base = (rounds - 1) * nbuf
            for b in range(nbuf):
                pltpu.make_async_copy(table_hbm.at[src_v.at[base + b]],
                                      rows_v.at[b], gsem.at[b]).wait()
                pltpu.async_copy(rows_v.at[b], acc.at[dst_v.at[base + b]],
                                 ssem.at[b], add=True)
            for b in range(nbuf):
                pltpu.make_async_copy(rows_v.at[b], acc.at[dst_v.at[base + b]],
                                      ssem.at[b]).wait()
            plsc.subcore_barrier()
            pltpu.sync_copy(acc.at[pl.ds(r0, rows_per_tile)],
                            out_hbm.at[c, pl.ds(r0, rows_per_tile)])

    return agg_kernel


def _mm0_body(x_ref, w_ref, deg_ref, y_ref, dinv_ref):
    deg = deg_ref[:, 0:1] + 1.0
    dinv = lax.rsqrt(deg)
    y = jnp.dot(x_ref[:], w_ref[:], preferred_element_type=jnp.float32)
    y_ref[:] = y * dinv
    dinv_ref[:] = dinv


def _mid_body(p0_ref, p1_ref, y0_ref, dinv_ref, b_ref, w_ref, out_ref):
    dinv = dinv_ref[:]
    pre = (p0_ref[0] + p1_ref[0] + y0_ref[:]) * dinv + b_ref[:]
    h = jnp.where(pre >= 0, pre, 0.01 * pre)
    out_ref[:] = jnp.dot(h, w_ref[:], preferred_element_type=jnp.float32) * dinv


def _make_final_body(num_graphs, n, bn):
    def final_body(p0_ref, p1_ref, y1_ref, dinv_ref, b_ref, bat_ref, out_ref):
        i = pl.program_id(0)

        @pl.when(i == 0)
        def _():
            out_ref[:] = jnp.full(out_ref.shape, -jnp.inf, jnp.float32)

        pre = (p0_ref[0] + p1_ref[0] + y1_ref[:]) * dinv_ref[:] + b_ref[:]
        h = jnp.where(pre >= 0, pre, 0.01 * pre)
        rid = jax.lax.broadcasted_iota(jnp.int32, (bn, 1), 0) + i * bn
        h = jnp.where(rid < n, h, -jnp.inf)
        bat = jnp.where(rid < n, bat_ref[:], num_graphs - 1)
        g_lo = jnp.clip(bat[0, 0], 0, num_graphs - 1)
        g_hi = jnp.clip(bat[bn - 1, 0], g_lo, num_graphs - 1)

        def body(g, carry):
            m = jnp.max(jnp.where(bat == g, h, -jnp.inf), axis=0,
                        keepdims=True)
            out_ref[pl.ds(g, 1), :] = jnp.maximum(out_ref[pl.ds(g, 1), :], m)
            return carry

        lax.fori_loop(g_lo, g_hi + 1, body, 0)

    return final_body


def _row_spec(bn, width):
    return pl.BlockSpec((bn, width), lambda i: (i, 0))


def _full_spec(shape):
    return pl.BlockSpec(shape, lambda i: tuple(0 for _ in shape))


def kernel(x, edge_index, batch, W0, b0, W1, b1):
    n, in_ch = x.shape
    hid = W0.shape[1]
    out_ch = W1.shape[1]
    e = edge_index.shape[1]
    num_graphs = 64

    bn = 1024
    n_pad = ((n + 1 + NS * 16 - 1) // (NS * 16)) * (NS * 16)
    n_pad = ((n_pad + bn - 1) // bn) * bn
    e_pad = ((e + NS * 8 * 128 - 1) // (NS * 8 * 128)) * (NS * 8 * 128)
    cpt_128 = e_pad // (128 * NS)
    tot_64 = e_pad // (64 * NS)
    cpt0_64 = min(int(round(tot_64 * 0.8 / 8)) * 8, tot_64 - 8)
    cpt1_64 = tot_64 - cpt0_64

    pad = jnp.full((e_pad - e,), n, jnp.int32)
    src_flat = jnp.concatenate([edge_index[0], pad])
    dst_flat = jnp.concatenate([edge_index[1], pad])
    src64 = src_flat.reshape(e_pad // 64, 64)
    dst64 = dst_flat.reshape(e_pad // 64, 64)
    dst128 = dst_flat.reshape(e_pad // 128, 128)

    ones_row = jnp.ones((128, 16), jnp.float32)
    z16 = jnp.zeros((n_pad, 16), jnp.float32)
    zh = jnp.zeros((n_pad, hid), jnp.float32)
    zo = jnp.zeros((n_pad, out_ch), jnp.float32)

    deg16 = _make_deg(n_pad, cpt_128)(dst128, ones_row, z16)

    grid = (n_pad // bn,)

    y0, dinv = pl.pallas_call(
        _mm0_body,
        grid=grid,
        in_specs=[
            _row_spec(bn, in_ch),
            _full_spec((in_ch, hid)),
            _row_spec(bn, 16),
        ],
        out_specs=[_row_spec(bn, hid), _row_spec(bn, 1)],
        out_shape=[
            jax.ShapeDtypeStruct((n_pad, hid), jnp.float32),
            jax.ShapeDtypeStruct((n_pad, 1), jnp.float32),
        ],
    )(x, W0, deg16)

    p = _make_agg(n_pad, hid, 64, cpt0_64, cpt1_64)(y0, src64, dst64, zh)

    y1 = pl.pallas_call(
        _mid_body,
        grid=grid,
        in_specs=[
            pl.BlockSpec((1, bn, hid), lambda i: (0, i, 0)),
            pl.BlockSpec((1, bn, hid), lambda i: (1, i, 0)),
            _row_spec(bn, hid),
            _row_spec(bn, 1),
            _full_spec((1, hid)),
            _full_spec((hid, out_ch)),
        ],
        out_specs=_row_spec(bn, out_ch),
        out_shape=jax.ShapeDtypeStruct((n_pad, out_ch), jnp.float32),
    )(p, p, y0, dinv, b0[None, :], W1)

    p2 = _make_agg(n_pad, out_ch, 64, cpt0_64, cpt1_64)(y1, src64, dst64, zo)

    out = pl.pallas_call(
        _make_final_body(num_graphs, n, bn),
        grid=grid,
        in_specs=[
            pl.BlockSpec((1, bn, out_ch), lambda i: (0, i, 0)),
            pl.BlockSpec((1, bn, out_ch), lambda i: (1, i, 0)),
            _row_spec(bn, out_ch),
            _row_spec(bn, 1),
            _full_spec((1, out_ch)),
            _row_spec(bn, 1),
        ],
        out_specs=_full_spec((num_graphs, out_ch)),
        out_shape=jax.ShapeDtypeStruct((num_graphs, out_ch), jnp.float32),
    )(p2, p2, y1, dinv, b1[None, :], batch[:, None])

    return out


# speedup vs baseline: 1.3760x; 1.2238x over previous
import functools

import jax
import jax.numpy as jnp
from jax import lax
from jax.experimental import pallas as pl
from jax.experimental.pallas import tpu as pltpu
from jax.experimental.pallas import tpu_sc as plsc

NC = 2
NS = 16


def _make_deg(n_pad, cpt):
    rows_per_tile = n_pad // NS
    mesh = plsc.VectorSubcoreMesh(core_axis_name="c", subcore_axis_name="s")

    @functools.partial(
        pl.kernel,
        out_type=jax.ShapeDtypeStruct((n_pad, 16), jnp.float32),
        mesh=mesh,
        scratch_types=[
            pltpu.VMEM((cpt, 128), jnp.int32),
            pltpu.VMEM((128, 16), jnp.float32),
            pltpu.VMEM_SHARED((n_pad, 16), jnp.float32),
            pltpu.SemaphoreType.DMA,
        ],
        compiler_params=pltpu.CompilerParams(use_tc_tiling_on_sc=False),
    )
    def deg_kernel(dst_hbm, ones_hbm, zeros_hbm, out_hbm, dst_v, ones_v, acc,
                   ssem):
        c = lax.axis_index("c")
        s = lax.axis_index("s")

        @pl.when(c == 0)
        def _():
            r0 = s * rows_per_tile
            pltpu.sync_copy(zeros_hbm.at[pl.ds(r0, rows_per_tile)],
                            acc.at[pl.ds(r0, rows_per_tile)])
            pltpu.sync_copy(dst_hbm.at[pl.ds(s * cpt, cpt)], dst_v)
            pltpu.sync_copy(ones_hbm, ones_v)
            plsc.subcore_barrier()

            def body(j, carry):
                pltpu.async_copy(ones_v, acc.at[dst_v.at[j]], ssem, add=True)
                return carry

            lax.fori_loop(0, cpt, body, 0)

            def drain(j, carry):
                pltpu.make_async_copy(ones_v, acc.at[dst_v.at[j]], ssem).wait()
                return carry

            lax.fori_loop(0, cpt, drain, 0)
            plsc.subcore_barrier()
            pltpu.sync_copy(acc.at[pl.ds(r0, rows_per_tile)],
                            out_hbm.at[pl.ds(r0, rows_per_tile)])

    return deg_kernel


def _make_agg(n_pad, width, ch, cpt0, cpt1):
    rows_per_tile = n_pad // NS
    mesh = plsc.VectorSubcoreMesh(core_axis_name="c", subcore_axis_name="s")

    nbuf = 8
    assert cpt0 % nbuf == 0 and cpt1 % nbuf == 0

    @functools.partial(
        pl.kernel,
        out_type=jax.ShapeDtypeStruct((NC, n_pad, width), jnp.float32),
        mesh=mesh,
        scratch_types=[
            pltpu.VMEM((cpt0, ch), jnp.int32),
            pltpu.VMEM((cpt0, ch), jnp.int32),
            pltpu.VMEM((nbuf, ch, width), jnp.float32),
            pltpu.VMEM_SHARED((n_pad, width), jnp.float32),
            pltpu.SemaphoreType.DMA((nbuf,)),
            pltpu.SemaphoreType.DMA((nbuf,)),
        ],
        compiler_params=pltpu.CompilerParams(use_tc_tiling_on_sc=False),
    )
    def agg_kernel(table_hbm, src_hbm, dst_hbm, zeros_hbm, out_hbm,
                   src_v, dst_v, rows_v, acc, gsem, ssem):
        c = lax.axis_index("c")
        s = lax.axis_index("s")
        rounds = jnp.where(c == 0, cpt0 // nbuf, cpt1 // nbuf)
        start = jnp.where(c == 0, s * cpt0, NS * cpt0 + s * cpt1)
        if True:
            r0 = s * rows_per_tile

            @pl.when(c == 0)
            def _():
                pltpu.sync_copy(src_hbm.at[pl.ds(start, cpt0)], src_v)
                pltpu.sync_copy(dst_hbm.at[pl.ds(start, cpt0)], dst_v)

            @pl.when(c != 0)
            def _():
                pltpu.sync_copy(src_hbm.at[pl.ds(start, cpt1)],
                                src_v.at[pl.ds(0, cpt1)])
                pltpu.sync_copy(dst_hbm.at[pl.ds(start, cpt1)],
                                dst_v.at[pl.ds(0, cpt1)])
            for b in range(nbuf):
                pltpu.async_copy(table_hbm.at[src_v.at[b]], rows_v.at[b],
                                 gsem.at[b])
            pltpu.sync_copy(zeros_hbm.at[pl.ds(r0, rows_per_tile)],
                            acc.at[pl.ds(r0, rows_per_tile)])
            plsc.subcore_barrier()

            def body(g, carry):
                base = g * nbuf
                for b in range(nbuf):
                    pltpu.make_async_copy(table_hbm.at[src_v.at[base + b]],
                                          rows_v.at[b], gsem.at[b]).wait()
                    pltpu.async_copy(rows_v.at[b], acc.at[dst_v.at[base + b]],
                                     ssem.at[b], add=True)
                for b in range(nbuf):
                    pltpu.make_async_copy(rows_v.at[b],
                                          acc.at[dst_v.at[base + b]],
                                          ssem.at[b]).wait()
                    pltpu.async_copy(table_hbm.at[src_v.at[base + nbuf + b]],
                                     rows_v.at[b], gsem.at[b])
                return carry

            lax.fori_loop(0, rounds - 1, body, 0)

            base = (rounds - 1) * nbuf
            for b in range(nbuf):
                pltpu.make_async_copy(table_hbm.at[src_v.at[base + b]],
                                      rows_v.at[b], gsem.at[b]).wait()
                pltpu.async_copy(rows_v.at[b], acc.at[dst_v.at[base + b]],
                                 ssem.at[b], add=True)
            for b in range(nbuf):
                pltpu.make_async_copy(rows_v.at[b], acc.at[dst_v.at[base + b]],
                                      ssem.at[b]).wait()
            plsc.subcore_barrier()
            pltpu.sync_copy(acc.at[pl.ds(r0, rows_per_tile)],
                            out_hbm.at[c, pl.ds(r0, rows_per_tile)])

    return agg_kernel


def _mm0_body(x_ref, w_ref, deg_ref, y_ref, dinv_ref):
    deg = deg_ref[:, 0:1] + 1.0
    dinv = lax.rsqrt(deg)
    y = jnp.dot(x_ref[:], w_ref[:], preferred_element_type=jnp.float32)
    y_ref[:] = y * dinv
    dinv_ref[:] = dinv


def _mid_body(p0_ref, p1_ref, y0_ref, dinv_ref, b_ref, w_ref, out_ref):
    dinv = dinv_ref[:]
    pre = (p0_ref[0] + p1_ref[0] + y0_ref[:]) * dinv + b_ref[:]
    h = jnp.where(pre >= 0, pre, 0.01 * pre)
    out_ref[:] = jnp.dot(h, w_ref[:], preferred_element_type=jnp.float32) * dinv


def _make_final_body(num_graphs, n, bn):
    def final_body(p0_ref, p1_ref, y1_ref, dinv_ref, b_ref, bat_ref, out_ref):
        i = pl.program_id(0)

        @pl.when(i == 0)
        def _():
            out_ref[:] = jnp.full(out_ref.shape, -jnp.inf, jnp.float32)

        pre = (p0_ref[0] + p1_ref[0] + y1_ref[:]) * dinv_ref[:] + b_ref[:]
        h = jnp.where(pre >= 0, pre, 0.01 * pre)
        rid = jax.lax.broadcasted_iota(jnp.int32, (bn, 1), 0) + i * bn
        h = jnp.where(rid < n, h, -jnp.inf)
        bat = jnp.where(rid < n, bat_ref[:], num_graphs - 1)
        g_lo = jnp.clip(bat[0, 0], 0, num_graphs - 1)
        g_hi = jnp.clip(bat[bn - 1, 0], g_lo, num_graphs - 1)

        def body(g, carry):
            m = jnp.max(jnp.where(bat == g, h, -jnp.inf), axis=0,
                        keepdims=True)
            out_ref[pl.ds(g, 1), :] = jnp.maximum(out_ref[pl.ds(g, 1), :], m)
            return carry

        lax.fori_loop(g_lo, g_hi + 1, body, 0)

    return final_body


def _row_spec(bn, width):
    return pl.BlockSpec((bn, width), lambda i: (i, 0))


def _full_spec(shape):
    return pl.BlockSpec(shape, lambda i: tuple(0 for _ in shape))


def kernel(x, edge_index, batch, W0, b0, W1, b1):
    n, in_ch = x.shape
    hid = W0.shape[1]
    out_ch = W1.shape[1]
    e = edge_index.shape[1]
    num_graphs = 64

    bn = 1024
    n_pad = ((n + 1 + NS * 16 - 1) // (NS * 16)) * (NS * 16)
    n_pad = ((n_pad + bn - 1) // bn) * bn
    e_pad = ((e + NS * 8 * 128 - 1) // (NS * 8 * 128)) * (NS * 8 * 128)
    cpt_128 = e_pad // (128 * NS)
    tot_64 = e_pad // (64 * NS)
    cpt0_64 = min(int(round(tot_64 * 0.9 / 8)) * 8, tot_64 - 8)
    cpt1_64 = tot_64 - cpt0_64

    pad = jnp.full((e_pad - e,), n, jnp.int32)
    src_flat = jnp.concatenate([edge_index[0], pad])
    dst_flat = jnp.concatenate([edge_index[1], pad])
    src64 = src_flat.reshape(e_pad // 64, 64)
    dst64 = dst_flat.reshape(e_pad // 64, 64)
    dst128 = dst_flat.reshape(e_pad // 128, 128)

    ones_row = jnp.ones((128, 16), jnp.float32)
    z16 = jnp.zeros((n_pad, 16), jnp.float32)
    zh = jnp.zeros((n_pad, hid), jnp.float32)
    zo = jnp.zeros((n_pad, out_ch), jnp.float32)

    deg16 = _make_deg(n_pad, cpt_128)(dst128, ones_row, z16)

    grid = (n_pad // bn,)

    y0, dinv = pl.pallas_call(
        _mm0_body,
        grid=grid,
        in_specs=[
            _row_spec(bn, in_ch),
            _full_spec((in_ch, hid)),
            _row_spec(bn, 16),
        ],
        out_specs=[_row_spec(bn, hid), _row_spec(bn, 1)],
        out_shape=[
            jax.ShapeDtypeStruct((n_pad, hid), jnp.float32),
            jax.ShapeDtypeStruct((n_pad, 1), jnp.float32),
        ],
    )(x, W0, deg16)

    p = _make_agg(n_pad, hid, 64, cpt0_64, cpt1_64)(y0, src64, dst64, zh)

    y1 = pl.pallas_call(
        _mid_body,
        grid=grid,
        in_specs=[
            pl.BlockSpec((1, bn, hid), lambda i: (0, i, 0)),
            pl.BlockSpec((1, bn, hid), lambda i: (1, i, 0)),
            _row_spec(bn, hid),
            _row_spec(bn, 1),
            _full_spec((1, hid)),
            _full_spec((hid, out_ch)),
        ],
        out_specs=_row_spec(bn, out_ch),
        out_shape=jax.ShapeDtypeStruct((n_pad, out_ch), jnp.float32),
    )(p, p, y0, dinv, b0[None, :], W1)

    p2 = _make_agg(n_pad, out_ch, 64, cpt0_64, cpt1_64)(y1, src64, dst64, zo)

    out = pl.pallas_call(
        _make_final_body(num_graphs, n, bn),
        grid=grid,
        in_specs=[
            pl.BlockSpec((1, bn, out_ch), lambda i: (0, i, 0)),
            pl.BlockSpec((1, bn, out_ch), lambda i: (1, i, 0)),
            _row_spec(bn, out_ch),
            _row_spec(bn, 1),
            _full_spec((1, out_ch)),
            _row_spec(bn, 1),
        ],
        out_specs=_full_spec((num_graphs, out_ch)),
        out_shape=jax.ShapeDtypeStruct((num_graphs, out_ch), jnp.float32),
    )(p2, p2, y1, dinv, b1[None, :], batch[:, None])

    return out


# speedup vs baseline: 1.3920x; 1.0116x over previous
import functools

import jax
import jax.numpy as jnp
from jax import lax
from jax.experimental import pallas as pl
from jax.experimental.pallas import tpu as pltpu
from jax.experimental.pallas import tpu_sc as plsc

NC = 2
NS = 16


def _make_deg(n_pad, cpt):
    rows_per_tile = n_pad // NS
    mesh = plsc.VectorSubcoreMesh(core_axis_name="c", subcore_axis_name="s")

    @functools.partial(
        pl.kernel,
        out_type=jax.ShapeDtypeStruct((n_pad, 16), jnp.float32),
        mesh=mesh,
        scratch_types=[
            pltpu.VMEM((cpt, 128), jnp.int32),
            pltpu.VMEM((128, 16), jnp.float32),
            pltpu.VMEM_SHARED((n_pad, 16), jnp.float32),
            pltpu.SemaphoreType.DMA,
        ],
        compiler_params=pltpu.CompilerParams(use_tc_tiling_on_sc=False),
    )
    def deg_kernel(dst_hbm, ones_hbm, zeros_hbm, out_hbm, dst_v, ones_v, acc,
                   ssem):
        c = lax.axis_index("c")
        s = lax.axis_index("s")

        @pl.when(c == 0)
        def _():
            r0 = s * rows_per_tile
            pltpu.sync_copy(zeros_hbm.at[pl.ds(r0, rows_per_tile)],
                            acc.at[pl.ds(r0, rows_per_tile)])
            pltpu.sync_copy(dst_hbm.at[pl.ds(s * cpt, cpt)], dst_v)
            pltpu.sync_copy(ones_hbm, ones_v)
            plsc.subcore_barrier()

            def body(j, carry):
                pltpu.async_copy(ones_v, acc.at[dst_v.at[j]], ssem, add=True)
                return carry

            lax.fori_loop(0, cpt, body, 0)

            def drain(j, carry):
                pltpu.make_async_copy(ones_v, acc.at[dst_v.at[j]], ssem).wait()
                return carry

            lax.fori_loop(0, cpt, drain, 0)
            plsc.subcore_barrier()
            pltpu.sync_copy(acc.at[pl.ds(r0, rows_per_tile)],
                            out_hbm.at[pl.ds(r0, rows_per_tile)])

    return deg_kernel


def _make_agg(n_pad, width, ch, cpt0, cpt1):
    rows_per_tile = n_pad // NS
    mesh = plsc.VectorSubcoreMesh(core_axis_name="c", subcore_axis_name="s")

    nbuf = 8
    assert cpt0 % nbuf == 0 and cpt1 % nbuf == 0

    @functools.partial(
        pl.kernel,
        out_type=jax.ShapeDtypeStruct((NC, n_pad, width), jnp.float32),
        mesh=mesh,
        scratch_types=[
            pltpu.VMEM((cpt0, ch), jnp.int32),
            pltpu.VMEM((cpt0, ch), jnp.int32),
            pltpu.VMEM((nbuf, ch, width), jnp.float32),
            pltpu.VMEM_SHARED((n_pad, width), jnp.float32),
            pltpu.SemaphoreType.DMA((nbuf,)),
            pltpu.SemaphoreType.DMA((nbuf,)),
        ],
        compiler_params=pltpu.CompilerParams(use_tc_tiling_on_sc=False),
    )
    def agg_kernel(table_hbm, src_hbm, dst_hbm, zeros_hbm, out_hbm,
                   src_v, dst_v, rows_v, acc, gsem, ssem):
        c = lax.axis_index("c")
        s = lax.axis_index("s")
        rounds = jnp.where(c == 0, cpt0 // nbuf, cpt1 // nbuf)
        start = jnp.where(c == 0, s * cpt0, NS * cpt0 + s * cpt1)
        if True:
            r0 = s * rows_per_tile

            @pl.when(c == 0)
            def _():
                pltpu.sync_copy(src_hbm.at[pl.ds(start, cpt0)], src_v)
                pltpu.sync_copy(dst_hbm.at[pl.ds(start, cpt0)], dst_v)

            @pl.when(c != 0)
            def _():
                pltpu.sync_copy(src_hbm.at[pl.ds(start, cpt1)],
                                src_v.at[pl.ds(0, cpt1)])
                pltpu.sync_copy(dst_hbm.at[pl.ds(start, cpt1)],
                                dst_v.at[pl.ds(0, cpt1)])
            for b in range(nbuf):
                pltpu.async_copy(table_hbm.at[src_v.at[b]], rows_v.at[b],
                                 gsem.at[b])
            pltpu.sync_copy(zeros_hbm.at[pl.ds(r0, rows_per_tile)],
                            acc.at[pl.ds(r0, rows_per_tile)])
            plsc.subcore_barrier()

            def body(g, carry):
                base = g * nbuf
                for b in range(nbuf):
                    pltpu.make_async_copy(table_hbm.at[src_v.at[base + b]],
                                          rows_v.at[b], gsem.at[b]).wait()
                    pltpu.async_copy(rows_v.at[b], acc.at[dst_v.at[base + b]],
                                     ssem.at[b], add=True)
                for b in range(nbuf):
                    pltpu.make_async_copy(rows_v.at[b],
                                          acc.at[dst_v.at[base + b]],
                                          ssem.at[b]).wait()
                    pltpu.async_copy(table_hbm.at[src_v.at[base + nbuf + b]],
                                     rows_v.at[b], gsem.at[b])
                return carry

            lax.fori_loop(0, rounds - 1, body, 0)

            base = (rounds - 1) * nbuf
            for b in range(nbuf):
                pltpu.make_async_copy(table_hbm.at[src_v.at[base + b]],
                                      rows_v.at[b], gsem.at[b]).wait()
                pltpu.async_copy(rows_v.at[b], acc.at[dst_v.at[base + b]],
                                 ssem.at[b], add=True)
            for b in range(nbuf):
                pltpu.make_async_copy(rows_v.at[b], acc.at[dst_v.at[base + b]],
                                      ssem.at[b]).wait()
            plsc.subcore_barrier()
            pltpu.sync_copy(acc.at[pl.ds(r0, rows_per_tile)],
                            out_hbm.at[c, pl.ds(r0, rows_per_tile)])

    return agg_kernel


def _mm0_body(x_ref, w_ref, deg_ref, y_ref, dinv_ref):
    deg = deg_ref[:, 0:1] + 1.0
    dinv = lax.rsqrt(deg)
    y = jnp.dot(x_ref[:], w_ref[:], preferred_element_type=jnp.float32)
    y_ref[:] = y * dinv
    dinv_ref[:] = dinv


def _mid_body(p0_ref, p1_ref, y0_ref, dinv_ref, b_ref, w_ref, out_ref):
    dinv = dinv_ref[:]
    pre = (p0_ref[0] + p1_ref[0] + y0_ref[:]) * dinv + b_ref[:]
    h = jnp.where(pre >= 0, pre, 0.01 * pre)
    out_ref[:] = jnp.dot(h, w_ref[:], preferred_element_type=jnp.float32) * dinv


def _make_final_body(num_graphs, n, bn):
    def final_body(p0_ref, p1_ref, y1_ref, dinv_ref, b_ref, bat_ref, out_ref):
        i = pl.program_id(0)

        @pl.when(i == 0)
        def _():
            out_ref[:] = jnp.full(out_ref.shape, -jnp.inf, jnp.float32)

        pre = (p0_ref[0] + p1_ref[0] + y1_ref[:]) * dinv_ref[:] + b_ref[:]
        h = jnp.where(pre >= 0, pre, 0.01 * pre)
        rid = jax.lax.broadcasted_iota(jnp.int32, (bn, 1), 0) + i * bn
        h = jnp.where(rid < n, h, -jnp.inf)
        bat = jnp.where(rid < n, bat_ref[:], num_graphs - 1)
        g_lo = jnp.clip(bat[0, 0], 0, num_graphs - 1)
        g_hi = jnp.clip(bat[bn - 1, 0], g_lo, num_graphs - 1)

        def body(g, carry):
            m = jnp.max(jnp.where(bat == g, h, -jnp.inf), axis=0,
                        keepdims=True)
            out_ref[pl.ds(g, 1), :] = jnp.maximum(out_ref[pl.ds(g, 1), :], m)
            return carry

        lax.fori_loop(g_lo, g_hi + 1, body, 0)

    return final_body


def _row_spec(bn, width):
    return pl.BlockSpec((bn, width), lambda i: (i, 0))


def _full_spec(shape):
    return pl.BlockSpec(shape, lambda i: tuple(0 for _ in shape))


def kernel(x, edge_index, batch, W0, b0, W1, b1):
    n, in_ch = x.shape
    hid = W0.shape[1]
    out_ch = W1.shape[1]
    e = edge_index.shape[1]
    num_graphs = 64

    bn = 1024
    n_pad = ((n + 1 + NS * 16 - 1) // (NS * 16)) * (NS * 16)
    n_pad = ((n_pad + bn - 1) // bn) * bn
    e_pad = ((e + NS * 8 * 128 - 1) // (NS * 8 * 128)) * (NS * 8 * 128)
    cpt_128 = e_pad // (128 * NS)
    tot_64 = e_pad // (64 * NS)
    cpt0_64 = min(int(round(tot_64 * 0.95 / 8)) * 8, tot_64 - 8)
    cpt1_64 = tot_64 - cpt0_64

    pad = jnp.full((e_pad - e,), n, jnp.int32)
    src_flat = jnp.concatenate([edge_index[0], pad])
    dst_flat = jnp.concatenate([edge_index[1], pad])
    src64 = src_flat.reshape(e_pad // 64, 64)
    dst64 = dst_flat.reshape(e_pad // 64, 64)
    dst128 = dst_flat.reshape(e_pad // 128, 128)

    ones_row = jnp.ones((128, 16), jnp.float32)
    z16 = jnp.zeros((n_pad, 16), jnp.float32)
    zh = jnp.zeros((n_pad, hid), jnp.float32)
    zo = jnp.zeros((n_pad, out_ch), jnp.float32)

    deg16 = _make_deg(n_pad, cpt_128)(dst128, ones_row, z16)

    grid = (n_pad // bn,)

    y0, dinv = pl.pallas_call(
        _mm0_body,
        grid=grid,
        in_specs=[
            _row_spec(bn, in_ch),
            _full_spec((in_ch, hid)),
            _row_spec(bn, 16),
        ],
        out_specs=[_row_spec(bn, hid), _row_spec(bn, 1)],
        out_shape=[
            jax.ShapeDtypeStruct((n_pad, hid), jnp.float32),
            jax.ShapeDtypeStruct((n_pad, 1), jnp.float32),
        ],
    )(x, W0, deg16)

    p = _make_agg(n_pad, hid, 64, cpt0_64, cpt1_64)(y0, src64, dst64, zh)

    y1 = pl.pallas_call(
        _mid_body,
        grid=grid,
        in_specs=[
            pl.BlockSpec((1, bn, hid), lambda i: (0, i, 0)),
            pl.BlockSpec((1, bn, hid), lambda i: (1, i, 0)),
            _row_spec(bn, hid),
            _row_spec(bn, 1),
            _full_spec((1, hid)),
            _full_spec((hid, out_ch)),
        ],
        out_specs=_row_spec(bn, out_ch),
        out_shape=jax.ShapeDtypeStruct((n_pad, out_ch), jnp.float32),
    )(p, p, y0, dinv, b0[None, :], W1)

    p2 = _make_agg(n_pad, out_ch, 64, cpt0_64, cpt1_64)(y1, src64, dst64, zo)

    out = pl.pallas_call(
        _make_final_body(num_graphs, n, bn),
        grid=grid,
        in_specs=[
            pl.BlockSpec((1, bn, out_ch), lambda i: (0, i, 0)),
            pl.BlockSpec((1, bn, out_ch), lambda i: (1, i, 0)),
            _row_spec(bn, out_ch),
            _row_spec(bn, 1),
            _full_spec((1, out_ch)),
            _row_spec(bn, 1),
        ],
        out_specs=_full_spec((num_graphs, out_ch)),
        out_shape=jax.ShapeDtypeStruct((num_graphs, out_ch), jnp.float32),
    )(p2, p2, y1, dinv, b1[None, :], batch[:, None])

    return out
